# baseline (speedup 1.0000x reference)
import functools

import jax
import jax.numpy as jnp
from jax import lax
from jax.experimental import pallas as pl
from jax.experimental.pallas import tpu as pltpu
from jax.experimental.pallas import tpu_sc as plsc

N_NODES = 10000
N_EDGES = 320000
D = 128

NC, NS = 2, 16
CHUNK = 128
CHUNKS = 80
N_PAD = 10240
E_PAD = NC * NS * CHUNKS * CHUNK
G = 8
TCH = NC * CHUNKS
NSTAGE = TCH // G
C0 = 32
RPT = N_PAD // NS


def _deg_body(dst_hbm, ones_hbm, zeros_hbm, deg_out, dst_v, ones_v, deg_sh, sem):
  c = lax.axis_index("c")
  s = lax.axis_index("s")
  pltpu.sync_copy(dst_hbm.at[c, s], dst_v)
  pltpu.sync_copy(ones_hbm, ones_v)
  pltpu.sync_copy(zeros_hbm, deg_sh.at[pl.ds(s * RPT, RPT)])
  plsc.subcore_barrier()

  def body(j, carry):
    pltpu.sync_copy(ones_v, deg_sh.at[dst_v.at[j]], add=True)
    return carry

  lax.fori_loop(0, CHUNKS, body, 0)
  plsc.subcore_barrier()
  pltpu.sync_copy(deg_sh.at[pl.ds(s * RPT, RPT)],
                  deg_out.at[c, pl.ds(s * RPT, RPT)])


NBUF = 2


def _scatter_body(xs_hbm, edges_hbm, zeros_hbm, part_out,
                  ib0, ib1, r0, r1, acc_sh, isem, sem0, sem1):
  c = lax.axis_index("c")
  s = lax.axis_index("s")
  ib = (ib0, ib1)
  rows = (r0, r1)
  sems = (sem0, sem1)

  c0s = C0 // G
  tstart = jnp.where(c == 0, 0, c0s)
  tend = jnp.where(c == 0, c0s, NSTAGE)

  pltpu.sync_copy(zeros_hbm, acc_sh.at[pl.ds(s * RPT, RPT)])
  plsc.subcore_barrier()

  pltpu.sync_copy(edges_hbm.at[s, pl.ds(tstart * G, G)], ib0)
  pltpu.async_copy(edges_hbm.at[s, pl.ds((tstart + 1) * G, G)], ib1, isem)
  pltpu.async_copy(xs_hbm.at[ib0.at[0, 0]], r0, sem0)
  pltpu.async_copy(xs_hbm.at[ib0.at[1, 0]], r1, sem1)

  def stage(t, par, last_pred, pref_pred):
    cur = ib[par]
    nxt = ib[1 - par]
    for g in range(G):
      k = g % 2
      pltpu.make_async_copy(xs_hbm.at[cur.at[g, 0]], rows[k],
                            sems[k]).wait()
      pltpu.sync_copy(rows[k], acc_sh.at[cur.at[g, 1]], add=True)
      if g < G - 2:
        pltpu.async_copy(xs_hbm.at[cur.at[g + 2, 0]], rows[k], sems[k])
      elif g == G - 2:
        @pl.when(last_pred)
        def _():
          pltpu.make_async_copy(edges_hbm.at[s, pl.ds(0, G)], nxt,
                                isem).wait()
          pltpu.async_copy(xs_hbm.at[nxt.at[0, 0]], rows[k], sems[k])
      else:
        @pl.when(last_pred)
        def _():
          pltpu.async_copy(xs_hbm.at[nxt.at[1, 0]], rows[k], sems[k])

    @pl.when(pref_pred)
    def _():
      pltpu.async_copy(edges_hbm.at[s, pl.ds((t + 2) * G, G)], cur, isem)

  def outer(t2, carry):
    te = tstart + 2 * t2
    to = te + 1
    stage(te, 0, te < tend - 1, te < tend - 2)
    stage(to, 1, to < tend - 1, to < tend - 2)
    return carry

  lax.fori_loop(0, (tend - tstart) // 2, outer, 0)
  plsc.subcore_barrier()
  pltpu.sync_copy(acc_sh.at[pl.ds(s * RPT, RPT)],
                  part_out.at[c, pl.ds(s * RPT, RPT)])


def _matmul_scale_body(mx_ref, w_ref, degp_ref, out_ref):
  i = pl.program_id(0)
  bm = out_ref.shape[0]
  x = jnp.dot(mx_ref[...], w_ref[...], preferred_element_type=jnp.float32)
  deg = (degp_ref[0, pl.ds(i * bm, bm)] + degp_ref[1, pl.ds(i * bm, bm)]
         + 1.0)
  out_ref[...] = x * lax.rsqrt(deg)[:, None]


def _finalize_body(p_ref, xs_ref, degp_ref, b_ref, out_ref):
  i = pl.program_id(0)
  bm = out_ref.shape[0]
  total = p_ref[0] + p_ref[1] + xs_ref[...]
  deg = (degp_ref[0, pl.ds(i * bm, bm)] + degp_ref[1, pl.ds(i * bm, bm)]
         + 1.0)
  out_ref[...] = total * lax.rsqrt(deg)[:, None] + b_ref[...][None, :]


def kernel(meta_xs, node_type, edge_index, edge_type, edge_time, W, b):
  del node_type, edge_type, edge_time

  src = edge_index[0].astype(jnp.int32)
  dst = edge_index[1].astype(jnp.int32)
  pad = E_PAD - N_EDGES
  src = jnp.concatenate([src, jnp.full((pad,), N_NODES, jnp.int32)])
  dst = jnp.concatenate([dst, jnp.full((pad,), N_NODES, jnp.int32)])
  dst4 = dst.reshape(NC, NS, CHUNKS, CHUNK)
  edges = jnp.stack([src.reshape(NS, TCH, CHUNK),
                     dst.reshape(NS, TCH, CHUNK)], axis=2)

  mx_pad = jnp.zeros((N_PAD, D), jnp.float32).at[:N_NODES].set(meta_xs)
  ones_row = jnp.ones((CHUNK,), jnp.float32)
  zeros_1d = jnp.zeros((RPT,), jnp.float32)
  zeros_2d = jnp.zeros((RPT, D), jnp.float32)

  mesh = plsc.VectorSubcoreMesh(core_axis_name="c", subcore_axis_name="s")

  deg_kernel = pl.kernel(
      _deg_body,
      out_type=jax.ShapeDtypeStruct((NC, N_PAD), jnp.float32),
      mesh=mesh,
      scratch_types=[
          pltpu.VMEM((CHUNKS, CHUNK), jnp.int32),
          pltpu.VMEM((CHUNK,), jnp.float32),
          pltpu.VMEM_SHARED((N_PAD,), jnp.float32),
          pltpu.SemaphoreType.DMA,
      ],
  )
  degp = deg_kernel(dst4, ones_row, zeros_1d)

  grid_m = N_PAD // 1024
  xs_scaled = pl.pallas_call(
      _matmul_scale_body,
      grid=(grid_m,),
      in_specs=[
          pl.BlockSpec((1024, D), lambda i: (i, 0)),
          pl.BlockSpec((D, D), lambda i: (0, 0)),
          pl.BlockSpec((NC, N_PAD), lambda i: (0, 0)),
      ],
      out_specs=pl.BlockSpec((1024, D), lambda i: (i, 0)),
      out_shape=jax.ShapeDtypeStruct((N_PAD, D), jnp.float32),
  )(mx_pad, W, degp)

  scatter_kernel = pl.kernel(
      _scatter_body,
      out_type=jax.ShapeDtypeStruct((NC, N_PAD, D), jnp.float32),
      mesh=mesh,
      scratch_types=[
          pltpu.VMEM((G, 2, CHUNK), jnp.int32),
          pltpu.VMEM((G, 2, CHUNK), jnp.int32),
          pltpu.VMEM((CHUNK, D), jnp.float32),
          pltpu.VMEM((CHUNK, D), jnp.float32),
          pltpu.VMEM_SHARED((N_PAD, D), jnp.float32),
          pltpu.SemaphoreType.DMA,
          pltpu.SemaphoreType.DMA,
          pltpu.SemaphoreType.DMA,
      ],
  )
  partials = scatter_kernel(xs_scaled, edges, zeros_2d)

  bm_out = 1024
  out = pl.pallas_call(
      _finalize_body,
      grid=(N_PAD // bm_out,),
      in_specs=[
          pl.BlockSpec((NC, bm_out, D), lambda i: (0, i, 0)),
          pl.BlockSpec((bm_out, D), lambda i: (i, 0)),
          pl.BlockSpec((NC, N_PAD), lambda i: (0, 0)),
          pl.BlockSpec((D,), lambda i: (0,)),
      ],
      out_specs=pl.BlockSpec((bm_out, D), lambda i: (i, 0)),
      out_shape=jax.ShapeDtypeStruct((N_PAD, D), jnp.float32),
  )(partials, xs_scaled, degp, b)
  return out[:N_NODES]


# speedup vs baseline: 1.1044x; 1.1044x over previous
import functools

import jax
import jax.numpy as jnp
from jax import lax
from jax.experimental import pallas as pl
from jax.experimental.pallas import tpu as pltpu
from jax.experimental.pallas import tpu_sc as plsc

N_NODES = 10000
N_EDGES = 320000
D = 128

NC, NS = 2, 16
CHUNK = 128
CHUNKS = 80
N_PAD = 10240
E_PAD = NC * NS * CHUNKS * CHUNK
G = 8
TCH = NC * CHUNKS
NSTAGE = TCH // G
C0 = 112
RPT = N_PAD // NS


def _deg_body(dst_hbm, ones_hbm, zeros_hbm, deg_out, dst_v, ones_v, deg_sh, sem):
  c = lax.axis_index("c")
  s = lax.axis_index("s")
  pltpu.sync_copy(dst_hbm.at[c, s], dst_v)
  pltpu.sync_copy(ones_hbm, ones_v)
  pltpu.sync_copy(zeros_hbm, deg_sh.at[pl.ds(s * RPT, RPT)])
  plsc.subcore_barrier()

  def body(j, carry):
    pltpu.sync_copy(ones_v, deg_sh.at[dst_v.at[j]], add=True)
    return carry

  lax.fori_loop(0, CHUNKS, body, 0)
  plsc.subcore_barrier()
  pltpu.sync_copy(deg_sh.at[pl.ds(s * RPT, RPT)],
                  deg_out.at[c, pl.ds(s * RPT, RPT)])


NBUF = 2


def _scatter_body(xs_hbm, edges_hbm, zeros_hbm, part_out,
                  ib0, ib1, r0, r1, acc_sh, isem, sem0, sem1):
  c = lax.axis_index("c")
  s = lax.axis_index("s")
  ib = (ib0, ib1)
  rows = (r0, r1)
  sems = (sem0, sem1)

  c0s = C0 // G
  tstart = jnp.where(c == 0, 0, c0s)
  tend = jnp.where(c == 0, c0s, NSTAGE)

  pltpu.sync_copy(zeros_hbm, acc_sh.at[pl.ds(s * RPT, RPT)])
  plsc.subcore_barrier()

  pltpu.sync_copy(edges_hbm.at[s, pl.ds(tstart * G, G)], ib0)
  pltpu.async_copy(edges_hbm.at[s, pl.ds((tstart + 1) * G, G)], ib1, isem)
  pltpu.async_copy(xs_hbm.at[ib0.at[0, 0]], r0, sem0)
  pltpu.async_copy(xs_hbm.at[ib0.at[1, 0]], r1, sem1)

  def stage(t, par, last_pred, pref_pred):
    cur = ib[par]
    nxt = ib[1 - par]
    for g in range(G):
      k = g % 2
      pltpu.make_async_copy(xs_hbm.at[cur.at[g, 0]], rows[k],
                            sems[k]).wait()
      pltpu.sync_copy(rows[k], acc_sh.at[cur.at[g, 1]], add=True)
      if g < G - 2:
        pltpu.async_copy(xs_hbm.at[cur.at[g + 2, 0]], rows[k], sems[k])
      elif g == G - 2:
        @pl.when(last_pred)
        def _():
          pltpu.make_async_copy(edges_hbm.at[s, pl.ds(0, G)], nxt,
                                isem).wait()
          pltpu.async_copy(xs_hbm.at[nxt.at[0, 0]], rows[k], sems[k])
      else:
        @pl.when(last_pred)
        def _():
          pltpu.async_copy(xs_hbm.at[nxt.at[1, 0]], rows[k], sems[k])

    @pl.when(pref_pred)
    def _():
      pltpu.async_copy(edges_hbm.at[s, pl.ds((t + 2) * G, G)], cur, isem)

  def outer(t2, carry):
    te = tstart + 2 * t2
    to = te + 1
    stage(te, 0, te < tend - 1, te < tend - 2)
    stage(to, 1, to < tend - 1, to < tend - 2)
    return carry

  lax.fori_loop(0, (tend - tstart) // 2, outer, 0)
  plsc.subcore_barrier()
  pltpu.sync_copy(acc_sh.at[pl.ds(s * RPT, RPT)],
                  part_out.at[c, pl.ds(s * RPT, RPT)])


def _matmul_scale_body(mx_ref, w_ref, degp_ref, out_ref):
  i = pl.program_id(0)
  bm = out_ref.shape[0]
  x = jnp.dot(mx_ref[...], w_ref[...], preferred_element_type=jnp.float32)
  deg = (degp_ref[0, pl.ds(i * bm, bm)] + degp_ref[1, pl.ds(i * bm, bm)]
         + 1.0)
  out_ref[...] = x * lax.rsqrt(deg)[:, None]


def _finalize_body(p_ref, xs_ref, degp_ref, b_ref, out_ref):
  i = pl.program_id(0)
  bm = out_ref.shape[0]
  total = p_ref[0] + p_ref[1] + xs_ref[...]
  deg = (degp_ref[0, pl.ds(i * bm, bm)] + degp_ref[1, pl.ds(i * bm, bm)]
         + 1.0)
  out_ref[...] = total * lax.rsqrt(deg)[:, None] + b_ref[...][None, :]


def kernel(meta_xs, node_type, edge_index, edge_type, edge_time, W, b):
  del node_type, edge_type, edge_time

  src = edge_index[0].astype(jnp.int32)
  dst = edge_index[1].astype(jnp.int32)
  pad = E_PAD - N_EDGES
  src = jnp.concatenate([src, jnp.full((pad,), N_NODES, jnp.int32)])
  dst = jnp.concatenate([dst, jnp.full((pad,), N_NODES, jnp.int32)])
  dst4 = dst.reshape(NC, NS, CHUNKS, CHUNK)
  edges = jnp.stack([src.reshape(NS, TCH, CHUNK),
                     dst.reshape(NS, TCH, CHUNK)], axis=2)

  mx_pad = jnp.zeros((N_PAD, D), jnp.float32).at[:N_NODES].set(meta_xs)
  ones_row = jnp.ones((CHUNK,), jnp.float32)
  zeros_1d = jnp.zeros((RPT,), jnp.float32)
  zeros_2d = jnp.zeros((RPT, D), jnp.float32)

  mesh = plsc.VectorSubcoreMesh(core_axis_name="c", subcore_axis_name="s")

  deg_kernel = pl.kernel(
      _deg_body,
      out_type=jax.ShapeDtypeStruct((NC, N_PAD), jnp.float32),
      mesh=mesh,
      scratch_types=[
          pltpu.VMEM((CHUNKS, CHUNK), jnp.int32),
          pltpu.VMEM((CHUNK,), jnp.float32),
          pltpu.VMEM_SHARED((N_PAD,), jnp.float32),
          pltpu.SemaphoreType.DMA,
      ],
  )
  degp = deg_kernel(dst4, ones_row, zeros_1d)

  grid_m = N_PAD // 1024
  xs_scaled = pl.pallas_call(
      _matmul_scale_body,
      grid=(grid_m,),
      in_specs=[
          pl.BlockSpec((1024, D), lambda i: (i, 0)),
          pl.BlockSpec((D, D), lambda i: (0, 0)),
          pl.BlockSpec((NC, N_PAD), lambda i: (0, 0)),
      ],
      out_specs=pl.BlockSpec((1024, D), lambda i: (i, 0)),
      out_shape=jax.ShapeDtypeStruct((N_PAD, D), jnp.float32),
  )(mx_pad, W, degp)

  scatter_kernel = pl.kernel(
      _scatter_body,
      out_type=jax.ShapeDtypeStruct((NC, N_PAD, D), jnp.float32),
      mesh=mesh,
      scratch_types=[
          pltpu.VMEM((G, 2, CHUNK), jnp.int32),
          pltpu.VMEM((G, 2, CHUNK), jnp.int32),
          pltpu.VMEM((CHUNK, D), jnp.float32),
          pltpu.VMEM((CHUNK, D), jnp.float32),
          pltpu.VMEM_SHARED((N_PAD, D), jnp.float32),
          pltpu.SemaphoreType.DMA,
          pltpu.SemaphoreType.DMA,
          pltpu.SemaphoreType.DMA,
      ],
  )
  partials = scatter_kernel(xs_scaled, edges, zeros_2d)

  bm_out = 1024
  out = pl.pallas_call(
      _finalize_body,
      grid=(N_PAD // bm_out,),
      in_specs=[
          pl.BlockSpec((NC, bm_out, D), lambda i: (0, i, 0)),
          pl.BlockSpec((bm_out, D), lambda i: (i, 0)),
          pl.BlockSpec((NC, N_PAD), lambda i: (0, 0)),
          pl.BlockSpec((D,), lambda i: (0,)),
      ],
      out_specs=pl.BlockSpec((bm_out, D), lambda i: (i, 0)),
      out_shape=jax.ShapeDtypeStruct((N_PAD, D), jnp.float32),
  )(partials, xs_scaled, degp, b)
  return out[:N_NODES]


# speedup vs baseline: 1.1135x; 1.0083x over previous
import functools

import jax
import jax.numpy as jnp
from jax import lax
from jax.experimental import pallas as pl
from jax.experimental.pallas import tpu as pltpu
from jax.experimental.pallas import tpu_sc as plsc

N_NODES = 10000
N_EDGES = 320000
D = 128

NC, NS = 2, 16
CHUNK = 128
CHUNKS = 80
N_PAD = 10240
E_PAD = NC * NS * CHUNKS * CHUNK
G = 8
TCH = NC * CHUNKS
NSTAGE = TCH // G
C0 = 112
RPT = N_PAD // NS


def _deg_body(dst_hbm, ones_hbm, zeros_hbm, deg_out, dst_v, ones_v, deg_sh, sem):
  c = lax.axis_index("c")
  s = lax.axis_index("s")
  pltpu.sync_copy(dst_hbm.at[c, s], dst_v)
  pltpu.sync_copy(ones_hbm, ones_v)
  pltpu.sync_copy(zeros_hbm, deg_sh.at[pl.ds(s * RPT, RPT)])
  plsc.subcore_barrier()

  def body(j, carry):
    pltpu.sync_copy(ones_v, deg_sh.at[dst_v.at[j]], add=True)
    return carry

  lax.fori_loop(0, CHUNKS, body, 0)
  plsc.subcore_barrier()
  pltpu.sync_copy(deg_sh.at[pl.ds(s * RPT, RPT)],
                  deg_out.at[c, pl.ds(s * RPT, RPT)])


NBUF = 2


def _scatter_body(xs_hbm, edges_hbm, part_out,
                  ib0, ib1, r0, r1, acc_sh, isem, sem0, sem1):
  c = lax.axis_index("c")
  s = lax.axis_index("s")
  ib = (ib0, ib1)
  rows = (r0, r1)
  sems = (sem0, sem1)

  c0s = C0 // G
  tstart = jnp.where(c == 0, 0, c0s)
  tend = jnp.where(c == 0, c0s, NSTAGE)

  zv = jnp.zeros((16,), jnp.float32)

  def zrow(i, carry):
    for q in range(D // 16):
      r0.at[i][pl.ds(q * 16, 16)] = zv
    return carry

  lax.fori_loop(0, CHUNK, zrow, 0)
  for rep in range(RPT // CHUNK):
    pltpu.sync_copy(r0, acc_sh.at[pl.ds(s * RPT + rep * CHUNK, CHUNK)])
  plsc.subcore_barrier()

  pltpu.sync_copy(edges_hbm.at[s, pl.ds(tstart * G, G)], ib0)
  pltpu.async_copy(edges_hbm.at[s, pl.ds((tstart + 1) * G, G)], ib1, isem)
  pltpu.async_copy(xs_hbm.at[ib0.at[0, 0]], r0, sem0)
  pltpu.async_copy(xs_hbm.at[ib0.at[1, 0]], r1, sem1)

  def stage(t, par, last_pred, pref_pred):
    cur = ib[par]
    nxt = ib[1 - par]
    for g in range(G):
      k = g % 2
      pltpu.make_async_copy(xs_hbm.at[cur.at[g, 0]], rows[k],
                            sems[k]).wait()
      pltpu.sync_copy(rows[k], acc_sh.at[cur.at[g, 1]], add=True)
      if g < G - 2:
        pltpu.async_copy(xs_hbm.at[cur.at[g + 2, 0]], rows[k], sems[k])
      elif g == G - 2:
        @pl.when(last_pred)
        def _():
          pltpu.make_async_copy(edges_hbm.at[s, pl.ds(0, G)], nxt,
                                isem).wait()
          pltpu.async_copy(xs_hbm.at[nxt.at[0, 0]], rows[k], sems[k])
      else:
        @pl.when(last_pred)
        def _():
          pltpu.async_copy(xs_hbm.at[nxt.at[1, 0]], rows[k], sems[k])

    @pl.when(pref_pred)
    def _():
      pltpu.async_copy(edges_hbm.at[s, pl.ds((t + 2) * G, G)], cur, isem)

  def outer(t2, carry):
    te = tstart + 2 * t2
    to = te + 1
    stage(te, 0, te < tend - 1, te < tend - 2)
    stage(to, 1, to < tend - 1, to < tend - 2)
    return carry

  lax.fori_loop(0, (tend - tstart) // 2, outer, 0)
  plsc.subcore_barrier()
  pltpu.sync_copy(acc_sh.at[pl.ds(s * RPT, RPT)],
                  part_out.at[c, pl.ds(s * RPT, RPT)])


def _matmul_scale_body(mx_ref, w_ref, degp_ref, out_ref):
  i = pl.program_id(0)
  bm = out_ref.shape[0]
  x = jnp.dot(mx_ref[...], w_ref[...], preferred_element_type=jnp.float32)
  deg = (degp_ref[0, pl.ds(i * bm, bm)] + degp_ref[1, pl.ds(i * bm, bm)]
         + 1.0)
  out_ref[...] = x * lax.rsqrt(deg)[:, None]


def _finalize_body(p_ref, xs_ref, degp_ref, b_ref, out_ref):
  i = pl.program_id(0)
  bm = out_ref.shape[0]
  total = p_ref[0] + p_ref[1] + xs_ref[...]
  deg = (degp_ref[0, pl.ds(i * bm, bm)] + degp_ref[1, pl.ds(i * bm, bm)]
         + 1.0)
  out_ref[...] = total * lax.rsqrt(deg)[:, None] + b_ref[...][None, :]


def kernel(meta_xs, node_type, edge_index, edge_type, edge_time, W, b):
  del node_type, edge_type, edge_time

  src = edge_index[0].astype(jnp.int32)
  dst = edge_index[1].astype(jnp.int32)
  pad = E_PAD - N_EDGES
  src = jnp.concatenate([src, jnp.full((pad,), N_NODES, jnp.int32)])
  dst = jnp.concatenate([dst, jnp.full((pad,), N_NODES, jnp.int32)])
  dst4 = dst.reshape(NC, NS, CHUNKS, CHUNK)
  edges = jnp.stack([src.reshape(NS, TCH, CHUNK),
                     dst.reshape(NS, TCH, CHUNK)], axis=2)

  mx_pad = jnp.zeros((N_PAD, D), jnp.float32).at[:N_NODES].set(meta_xs)
  ones_row = jnp.ones((CHUNK,), jnp.float32)
  zeros_1d = jnp.zeros((RPT,), jnp.float32)

  mesh = plsc.VectorSubcoreMesh(core_axis_name="c", subcore_axis_name="s")

  deg_kernel = pl.kernel(
      _deg_body,
      out_type=jax.ShapeDtypeStruct((NC, N_PAD), jnp.float32),
      mesh=mesh,
      scratch_types=[
          pltpu.VMEM((CHUNKS, CHUNK), jnp.int32),
          pltpu.VMEM((CHUNK,), jnp.float32),
          pltpu.VMEM_SHARED((N_PAD,), jnp.float32),
          pltpu.SemaphoreType.DMA,
      ],
  )
  degp = deg_kernel(dst4, ones_row, zeros_1d)

  grid_m = N_PAD // 1024
  xs_scaled = pl.pallas_call(
      _matmul_scale_body,
      grid=(grid_m,),
      in_specs=[
          pl.BlockSpec((1024, D), lambda i: (i, 0)),
          pl.BlockSpec((D, D), lambda i: (0, 0)),
          pl.BlockSpec((NC, N_PAD), lambda i: (0, 0)),
      ],
      out_specs=pl.BlockSpec((1024, D), lambda i: (i, 0)),
      out_shape=jax.ShapeDtypeStruct((N_PAD, D), jnp.float32),
  )(mx_pad, W, degp)

  scatter_kernel = pl.kernel(
      _scatter_body,
      out_type=jax.ShapeDtypeStruct((NC, N_PAD, D), jnp.float32),
      mesh=mesh,
      scratch_types=[
          pltpu.VMEM((G, 2, CHUNK), jnp.int32),
          pltpu.VMEM((G, 2, CHUNK), jnp.int32),
          pltpu.VMEM((CHUNK, D), jnp.float32),
          pltpu.VMEM((CHUNK, D), jnp.float32),
          pltpu.VMEM_SHARED((N_PAD, D), jnp.float32),
          pltpu.SemaphoreType.DMA,
          pltpu.SemaphoreType.DMA,
          pltpu.SemaphoreType.DMA,
      ],
  )
  partials = scatter_kernel(xs_scaled, edges)

  bm_out = 1024
  out = pl.pallas_call(
      _finalize_body,
      grid=(N_PAD // bm_out,),
      in_specs=[
          pl.BlockSpec((NC, bm_out, D), lambda i: (0, i, 0)),
          pl.BlockSpec((bm_out, D), lambda i: (i, 0)),
          pl.BlockSpec((NC, N_PAD), lambda i: (0, 0)),
          pl.BlockSpec((D,), lambda i: (0,)),
      ],
      out_specs=pl.BlockSpec((bm_out, D), lambda i: (i, 0)),
      out_shape=jax.ShapeDtypeStruct((N_PAD, D), jnp.float32),
  )(partials, xs_scaled, degp, b)
  return out[:N_NODES]


# speedup vs baseline: 1.1284x; 1.0134x over previous
import functools

import jax
import jax.numpy as jnp
from jax import lax
from jax.experimental import pallas as pl
from jax.experimental.pallas import tpu as pltpu
from jax.experimental.pallas import tpu_sc as plsc

N_NODES = 10000
N_EDGES = 320000
D = 128

NC, NS = 2, 16
CHUNK = 128
CHUNKS = 80
N_PAD = 10240
E_PAD = NC * NS * CHUNKS * CHUNK
G = 8
TCH = NC * CHUNKS
NSTAGE = TCH // G
C0 = 144
RPT = N_PAD // NS


def _deg_body(dst_hbm, ones_hbm, zeros_hbm, deg_out, dst_v, ones_v, deg_sh, sem):
  c = lax.axis_index("c")
  s = lax.axis_index("s")
  pltpu.sync_copy(dst_hbm.at[c, s], dst_v)
  pltpu.sync_copy(ones_hbm, ones_v)
  pltpu.sync_copy(zeros_hbm, deg_sh.at[pl.ds(s * RPT, RPT)])
  plsc.subcore_barrier()

  def body(j, carry):
    pltpu.sync_copy(ones_v, deg_sh.at[dst_v.at[j]], add=True)
    return carry

  lax.fori_loop(0, CHUNKS, body, 0)
  plsc.subcore_barrier()
  pltpu.sync_copy(deg_sh.at[pl.ds(s * RPT, RPT)],
                  deg_out.at[c, pl.ds(s * RPT, RPT)])


NBUF = 2


def _scatter_body(xs_hbm, edges_hbm, part_out,
                  ib0, ib1, r0, r1, acc_sh, isem, sem0, sem1):
  c = lax.axis_index("c")
  s = lax.axis_index("s")
  ib = (ib0, ib1)
  rows = (r0, r1)
  sems = (sem0, sem1)

  c0s = C0 // G
  tstart = jnp.where(c == 0, 0, c0s)
  tend = jnp.where(c == 0, c0s, NSTAGE)

  zv = jnp.zeros((16,), jnp.float32)

  def zrow(i, carry):
    for q in range(D // 16):
      r0.at[i][pl.ds(q * 16, 16)] = zv
    return carry

  lax.fori_loop(0, CHUNK, zrow, 0)
  for rep in range(RPT // CHUNK):
    pltpu.sync_copy(r0, acc_sh.at[pl.ds(s * RPT + rep * CHUNK, CHUNK)])
  plsc.subcore_barrier()

  pltpu.sync_copy(edges_hbm.at[s, pl.ds(tstart * G, G)], ib0)
  pltpu.async_copy(edges_hbm.at[s, pl.ds((tstart + 1) * G, G)], ib1, isem)
  pltpu.async_copy(xs_hbm.at[ib0.at[0, 0]], r0, sem0)
  pltpu.async_copy(xs_hbm.at[ib0.at[1, 0]], r1, sem1)

  def stage(t, par, last_pred, pref_pred):
    cur = ib[par]
    nxt = ib[1 - par]
    for g in range(G):
      k = g % 2
      pltpu.make_async_copy(xs_hbm.at[cur.at[g, 0]], rows[k],
                            sems[k]).wait()
      pltpu.sync_copy(rows[k], acc_sh.at[cur.at[g, 1]], add=True)
      if g < G - 2:
        pltpu.async_copy(xs_hbm.at[cur.at[g + 2, 0]], rows[k], sems[k])
      elif g == G - 2:
        @pl.when(last_pred)
        def _():
          pltpu.make_async_copy(edges_hbm.at[s, pl.ds(0, G)], nxt,
                                isem).wait()
          pltpu.async_copy(xs_hbm.at[nxt.at[0, 0]], rows[k], sems[k])
      else:
        @pl.when(last_pred)
        def _():
          pltpu.async_copy(xs_hbm.at[nxt.at[1, 0]], rows[k], sems[k])

    @pl.when(pref_pred)
    def _():
      pltpu.async_copy(edges_hbm.at[s, pl.ds((t + 2) * G, G)], cur, isem)

  def outer(t2, carry):
    te = tstart + 2 * t2
    to = te + 1
    stage(te, 0, te < tend - 1, te < tend - 2)
    stage(to, 1, to < tend - 1, to < tend - 2)
    return carry

  lax.fori_loop(0, (tend - tstart) // 2, outer, 0)
  plsc.subcore_barrier()
  pltpu.sync_copy(acc_sh.at[pl.ds(s * RPT, RPT)],
                  part_out.at[c, pl.ds(s * RPT, RPT)])


def _matmul_scale_body(mx_ref, w_ref, degp_ref, out_ref):
  i = pl.program_id(0)
  bm = out_ref.shape[0]
  x = jnp.dot(mx_ref[...], w_ref[...], preferred_element_type=jnp.float32)
  deg = (degp_ref[0, pl.ds(i * bm, bm)] + degp_ref[1, pl.ds(i * bm, bm)]
         + 1.0)
  out_ref[...] = x * lax.rsqrt(deg)[:, None]


def _finalize_body(p_ref, xs_ref, degp_ref, b_ref, out_ref):
  i = pl.program_id(0)
  bm = out_ref.shape[0]
  total = p_ref[0] + p_ref[1] + xs_ref[...]
  deg = (degp_ref[0, pl.ds(i * bm, bm)] + degp_ref[1, pl.ds(i * bm, bm)]
         + 1.0)
  out_ref[...] = total * lax.rsqrt(deg)[:, None] + b_ref[...][None, :]


def kernel(meta_xs, node_type, edge_index, edge_type, edge_time, W, b):
  del node_type, edge_type, edge_time

  src = edge_index[0].astype(jnp.int32)
  dst = edge_index[1].astype(jnp.int32)
  pad = E_PAD - N_EDGES
  src = jnp.concatenate([src, jnp.full((pad,), N_NODES, jnp.int32)])
  dst = jnp.concatenate([dst, jnp.full((pad,), N_NODES, jnp.int32)])
  dst4 = dst.reshape(NC, NS, CHUNKS, CHUNK)
  edges = jnp.stack([src.reshape(NS, TCH, CHUNK),
                     dst.reshape(NS, TCH, CHUNK)], axis=2)

  mx_pad = jnp.zeros((N_PAD, D), jnp.float32).at[:N_NODES].set(meta_xs)
  ones_row = jnp.ones((CHUNK,), jnp.float32)
  zeros_1d = jnp.zeros((RPT,), jnp.float32)

  mesh = plsc.VectorSubcoreMesh(core_axis_name="c", subcore_axis_name="s")

  deg_kernel = pl.kernel(
      _deg_body,
      out_type=jax.ShapeDtypeStruct((NC, N_PAD), jnp.float32),
      mesh=mesh,
      scratch_types=[
          pltpu.VMEM((CHUNKS, CHUNK), jnp.int32),
          pltpu.VMEM((CHUNK,), jnp.float32),
          pltpu.VMEM_SHARED((N_PAD,), jnp.float32),
          pltpu.SemaphoreType.DMA,
      ],
  )
  degp = deg_kernel(dst4, ones_row, zeros_1d)

  grid_m = N_PAD // 1024
  xs_scaled = pl.pallas_call(
      _matmul_scale_body,
      grid=(grid_m,),
      in_specs=[
          pl.BlockSpec((1024, D), lambda i: (i, 0)),
          pl.BlockSpec((D, D), lambda i: (0, 0)),
          pl.BlockSpec((NC, N_PAD), lambda i: (0, 0)),
      ],
      out_specs=pl.BlockSpec((1024, D), lambda i: (i, 0)),
      out_shape=jax.ShapeDtypeStruct((N_PAD, D), jnp.float32),
  )(mx_pad, W, degp)

  scatter_kernel = pl.kernel(
      _scatter_body,
      out_type=jax.ShapeDtypeStruct((NC, N_PAD, D), jnp.float32),
      mesh=mesh,
      scratch_types=[
          pltpu.VMEM((G, 2, CHUNK), jnp.int32),
          pltpu.VMEM((G, 2, CHUNK), jnp.int32),
          pltpu.VMEM((CHUNK, D), jnp.float32),
          pltpu.VMEM((CHUNK, D), jnp.float32),
          pltpu.VMEM_SHARED((N_PAD, D), jnp.float32),
          pltpu.SemaphoreType.DMA,
          pltpu.SemaphoreType.DMA,
          pltpu.SemaphoreType.DMA,
      ],
  )
  partials = scatter_kernel(xs_scaled, edges)

  bm_out = 1024
  out = pl.pallas_call(
      _finalize_body,
      grid=(N_PAD // bm_out,),
      in_specs=[
          pl.BlockSpec((NC, bm_out, D), lambda i: (0, i, 0)),
          pl.BlockSpec((bm_out, D), lambda i: (i, 0)),
          pl.BlockSpec((NC, N_PAD), lambda i: (0, 0)),
          pl.BlockSpec((D,), lambda i: (0,)),
      ],
      out_specs=pl.BlockSpec((bm_out, D), lambda i: (i, 0)),
      out_shape=jax.ShapeDtypeStruct((N_PAD, D), jnp.float32),
  )(partials, xs_scaled, degp, b)
  return out[:N_NODES]


# speedup vs baseline: 1.1305x; 1.0018x over previous
import functools

import jax
import jax.numpy as jnp
from jax import lax
from jax.experimental import pallas as pl
from jax.experimental.pallas import tpu as pltpu
from jax.experimental.pallas import tpu_sc as plsc

N_NODES = 10000
N_EDGES = 320000
D = 128

NC, NS = 2, 16
CHUNK = 128
CHUNKS = 80
N_PAD = 10240
E_PAD = NC * NS * CHUNKS * CHUNK
G = 8
TCH = NC * CHUNKS
NSTAGE = TCH // G
C0 = 128
RPT = N_PAD // NS


def _deg_body(dst_hbm, ones_hbm, zeros_hbm, deg_out, dst_v, ones_v, deg_sh, sem):
  c = lax.axis_index("c")
  s = lax.axis_index("s")
  pltpu.sync_copy(dst_hbm.at[c, s], dst_v)
  pltpu.sync_copy(ones_hbm, ones_v)
  pltpu.sync_copy(zeros_hbm, deg_sh.at[pl.ds(s * RPT, RPT)])
  plsc.subcore_barrier()

  def body(j, carry):
    pltpu.sync_copy(ones_v, deg_sh.at[dst_v.at[j]], add=True)
    return carry

  lax.fori_loop(0, CHUNKS, body, 0)
  plsc.subcore_barrier()
  pltpu.sync_copy(deg_sh.at[pl.ds(s * RPT, RPT)],
                  deg_out.at[c, pl.ds(s * RPT, RPT)])


NBUF = 2


def _scatter_body(xs_hbm, edges_hbm, part_out,
                  ib0, ib1, r0, r1, acc_sh, isem, sem0, sem1):
  c = lax.axis_index("c")
  s = lax.axis_index("s")
  ib = (ib0, ib1)
  rows = (r0, r1)
  sems = (sem0, sem1)

  c0s = C0 // G
  tstart = jnp.where(c == 0, 0, c0s)
  tend = jnp.where(c == 0, c0s, NSTAGE)

  zv = jnp.zeros((16,), jnp.float32)

  def zrow(i, carry):
    for q in range(D // 16):
      r0.at[i][pl.ds(q * 16, 16)] = zv
    return carry

  lax.fori_loop(0, CHUNK, zrow, 0)
  for rep in range(RPT // CHUNK):
    pltpu.sync_copy(r0, acc_sh.at[pl.ds(s * RPT + rep * CHUNK, CHUNK)])
  plsc.subcore_barrier()

  pltpu.sync_copy(edges_hbm.at[s, pl.ds(tstart * G, G)], ib0)
  pltpu.async_copy(edges_hbm.at[s, pl.ds((tstart + 1) * G, G)], ib1, isem)
  pltpu.async_copy(xs_hbm.at[ib0.at[0, 0]], r0, sem0)
  pltpu.async_copy(xs_hbm.at[ib0.at[1, 0]], r1, sem1)

  def stage(t, par, last_pred, pref_pred):
    cur = ib[par]
    nxt = ib[1 - par]
    for g in range(G):
      k = g % 2
      pltpu.make_async_copy(xs_hbm.at[cur.at[g, 0]], rows[k],
                            sems[k]).wait()
      pltpu.sync_copy(rows[k], acc_sh.at[cur.at[g, 1]], add=True)
      if g < G - 2:
        pltpu.async_copy(xs_hbm.at[cur.at[g + 2, 0]], rows[k], sems[k])
      elif g == G - 2:
        @pl.when(last_pred)
        def _():
          pltpu.make_async_copy(edges_hbm.at[s, pl.ds(0, G)], nxt,
                                isem).wait()
          pltpu.async_copy(xs_hbm.at[nxt.at[0, 0]], rows[k], sems[k])
      else:
        @pl.when(last_pred)
        def _():
          pltpu.async_copy(xs_hbm.at[nxt.at[1, 0]], rows[k], sems[k])

    @pl.when(pref_pred)
    def _():
      pltpu.async_copy(edges_hbm.at[s, pl.ds((t + 2) * G, G)], cur, isem)

  def outer(t2, carry):
    te = tstart + 2 * t2
    to = te + 1
    stage(te, 0, te < tend - 1, te < tend - 2)
    stage(to, 1, to < tend - 1, to < tend - 2)
    return carry

  lax.fori_loop(0, (tend - tstart) // 2, outer, 0)
  plsc.subcore_barrier()
  pltpu.sync_copy(acc_sh.at[pl.ds(s * RPT, RPT)],
                  part_out.at[c, pl.ds(s * RPT, RPT)])


def _matmul_scale_body(mx_ref, w_ref, degp_ref, out_ref):
  i = pl.program_id(0)
  bm = out_ref.shape[0]
  x = jnp.dot(mx_ref[...], w_ref[...], preferred_element_type=jnp.float32)
  deg = (degp_ref[0, pl.ds(i * bm, bm)] + degp_ref[1, pl.ds(i * bm, bm)]
         + 1.0)
  out_ref[...] = x * lax.rsqrt(deg)[:, None]


def _finalize_body(p_ref, xs_ref, degp_ref, b_ref, out_ref):
  i = pl.program_id(0)
  bm = out_ref.shape[0]
  total = p_ref[0] + p_ref[1] + xs_ref[...]
  deg = (degp_ref[0, pl.ds(i * bm, bm)] + degp_ref[1, pl.ds(i * bm, bm)]
         + 1.0)
  out_ref[...] = total * lax.rsqrt(deg)[:, None] + b_ref[...][None, :]


def kernel(meta_xs, node_type, edge_index, edge_type, edge_time, W, b):
  del node_type, edge_type, edge_time

  src = edge_index[0].astype(jnp.int32)
  dst = edge_index[1].astype(jnp.int32)
  pad = E_PAD - N_EDGES
  src = jnp.concatenate([src, jnp.full((pad,), N_NODES, jnp.int32)])
  dst = jnp.concatenate([dst, jnp.full((pad,), N_NODES, jnp.int32)])
  dst4 = dst.reshape(NC, NS, CHUNKS, CHUNK)
  edges = jnp.stack([src.reshape(NS, TCH, CHUNK),
                     dst.reshape(NS, TCH, CHUNK)], axis=2)

  mx_pad = jnp.zeros((N_PAD, D), jnp.float32).at[:N_NODES].set(meta_xs)
  ones_row = jnp.ones((CHUNK,), jnp.float32)
  zeros_1d = jnp.zeros((RPT,), jnp.float32)

  mesh = plsc.VectorSubcoreMesh(core_axis_name="c", subcore_axis_name="s")

  deg_kernel = pl.kernel(
      _deg_body,
      out_type=jax.ShapeDtypeStruct((NC, N_PAD), jnp.float32),
      mesh=mesh,
      scratch_types=[
          pltpu.VMEM((CHUNKS, CHUNK), jnp.int32),
          pltpu.VMEM((CHUNK,), jnp.float32),
          pltpu.VMEM_SHARED((N_PAD,), jnp.float32),
          pltpu.SemaphoreType.DMA,
      ],
  )
  degp = deg_kernel(dst4, ones_row, zeros_1d)

  grid_m = N_PAD // 1024
  xs_scaled = pl.pallas_call(
      _matmul_scale_body,
      grid=(grid_m,),
      in_specs=[
          pl.BlockSpec((1024, D), lambda i: (i, 0)),
          pl.BlockSpec((D, D), lambda i: (0, 0)),
          pl.BlockSpec((NC, N_PAD), lambda i: (0, 0)),
      ],
      out_specs=pl.BlockSpec((1024, D), lambda i: (i, 0)),
      out_shape=jax.ShapeDtypeStruct((N_PAD, D), jnp.float32),
  )(mx_pad, W, degp)

  scatter_kernel = pl.kernel(
      _scatter_body,
      out_type=jax.ShapeDtypeStruct((NC, N_PAD, D), jnp.float32),
      mesh=mesh,
      scratch_types=[
          pltpu.VMEM((G, 2, CHUNK), jnp.int32),
          pltpu.VMEM((G, 2, CHUNK), jnp.int32),
          pltpu.VMEM((CHUNK, D), jnp.float32),
          pltpu.VMEM((CHUNK, D), jnp.float32),
          pltpu.VMEM_SHARED((N_PAD, D), jnp.float32),
          pltpu.SemaphoreType.DMA,
          pltpu.SemaphoreType.DMA,
          pltpu.SemaphoreType.DMA,
      ],
  )
  partials = scatter_kernel(xs_scaled, edges)

  bm_out = 1024
  out = pl.pallas_call(
      _finalize_body,
      grid=(N_PAD // bm_out,),
      in_specs=[
          pl.BlockSpec((NC, bm_out, D), lambda i: (0, i, 0)),
          pl.BlockSpec((bm_out, D), lambda i: (i, 0)),
          pl.BlockSpec((NC, N_PAD), lambda i: (0, 0)),
          pl.BlockSpec((D,), lambda i: (0,)),
      ],
      out_specs=pl.BlockSpec((bm_out, D), lambda i: (i, 0)),
      out_shape=jax.ShapeDtypeStruct((N_PAD, D), jnp.float32),
  )(partials, xs_scaled, degp, b)
  return out[:N_NODES]
